# baseline (speedup 1.0000x reference)
import functools

import jax
import jax.numpy as jnp
from jax import lax
from jax.experimental import pallas as pl
from jax.experimental.pallas import tpu as pltpu
from jax.experimental.pallas import tpu_sc as plsc

_D = 128
_H = 8
_DH = _D // _H


def _sc_gather(table, idx, chunk=80):
    V, D = table.shape
    B = idx.shape[0]
    info = plsc.get_sparse_core_info()
    nc, ns = info.num_cores, info.num_subcores
    nw = nc * ns
    assert B % (nw * chunk) == 0, (B, nw, chunk)
    b_per_w = B // nw
    nchunks = b_per_w // chunk
    mesh = plsc.VectorSubcoreMesh(core_axis_name="c", subcore_axis_name="s")

    def body(table_hbm, idx_hbm, out_hbm, idx_v, rows_v, sem):
        wid = lax.axis_index("s") * nc + lax.axis_index("c")
        base = wid * b_per_w

        @pl.loop(0, nchunks)
        def _(t):
            off = base + t * chunk
            pltpu.sync_copy(idx_hbm.at[pl.ds(off, chunk)], idx_v)
            pltpu.async_copy(table_hbm.at[idx_v], rows_v, sem).wait()
            pltpu.sync_copy(rows_v, out_hbm.at[pl.ds(off, chunk)])

    return pl.kernel(
        body,
        out_type=jax.ShapeDtypeStruct((B, D), table.dtype),
        mesh=mesh,
        scratch_types=[
            pltpu.VMEM((chunk,), jnp.int32),
            pltpu.VMEM((chunk, D), table.dtype),
            pltpu.SemaphoreType.DMA,
        ],
    )(table, idx)


def _seg_reduce(offsets, vals, mode, n_seg, bl=16, c=256):
    R, Dv = vals.shape
    assert n_seg % bl == 0

    def kern(off_ref, vals_hbm, out_ref, scratch, sem):
        b = pl.program_id(0)
        lane0 = b * bl
        start = off_ref[lane0]
        end = off_ref[lane0 + bl]
        nch = (end - start + c - 1) // c
        if mode == "max":
            init = jnp.full((bl, Dv), -jnp.inf, jnp.float32)
        else:
            init = jnp.zeros((bl, Dv), jnp.float32)

        def chunk_body(t, acc):
            off_i = start + t * c
            off_r = jnp.minimum(off_i, R - c)
            cp = pltpu.make_async_copy(vals_hbm.at[pl.ds(off_r, c), :], scratch, sem)
            cp.start()
            cp.wait()
            v = scratch[...]
            gidx = off_r + lax.broadcasted_iota(jnp.int32, (c, 1), 0)
            valid = gidx >= off_i
            if mode == "max":
                rows = []
                for j in range(bl):
                    sj = off_ref[lane0 + j]
                    ej = off_ref[lane0 + j + 1]
                    m = (gidx >= sj) & (gidx < ej) & valid
                    contrib = jnp.where(m, v, -jnp.inf).max(axis=0, keepdims=True)
                    rows.append(jnp.maximum(acc[j:j + 1], contrib))
                return jnp.concatenate(rows, axis=0)
            else:
                cols = []
                for j in range(bl):
                    sj = off_ref[lane0 + j]
                    ej = off_ref[lane0 + j + 1]
                    m = (gidx >= sj) & (gidx < ej) & valid
                    cols.append(m.astype(jnp.float32))
                mask = jnp.concatenate(cols, axis=1)
                part = lax.dot_general(
                    mask, v, (((0,), (0,)), ((), ())),
                    preferred_element_type=jnp.float32)
                return acc + part

        acc = lax.fori_loop(0, nch, chunk_body, init)
        if mode == "max":
            acc = jnp.where(acc == -jnp.inf, 0.0, acc)
        out_ref[...] = acc

    grid_spec = pltpu.PrefetchScalarGridSpec(
        num_scalar_prefetch=1,
        grid=(n_seg // bl,),
        in_specs=[pl.BlockSpec(memory_space=pl.MemorySpace.ANY)],
        out_specs=pl.BlockSpec((bl, Dv), lambda b, off: (b, 0)),
        scratch_shapes=[pltpu.VMEM((c, Dv), jnp.float32), pltpu.SemaphoreType.DMA],
    )
    return pl.pallas_call(
        kern,
        grid_spec=grid_spec,
        out_shape=jax.ShapeDtypeStruct((n_seg, Dv), jnp.float32),
    )(offsets, vals)


def _rows(body, row_ins, full_ins, out_dims, br=1000):
    R = row_ins[0].shape[0]
    assert R % br == 0
    n_row = len(row_ins)

    def kern(*refs):
        ins = refs[: n_row + len(full_ins)]
        outs = refs[n_row + len(full_ins):]
        row_vals = [r[...] for r in ins[:n_row]]
        full_vals = [r[...] for r in ins[n_row:]]
        res = body(row_vals, full_vals)
        if not isinstance(res, tuple):
            res = (res,)
        for o_ref, r in zip(outs, res):
            o_ref[...] = r

    in_specs = [
        pl.BlockSpec((br, a.shape[1]), lambda i: (i, 0)) for a in row_ins
    ] + [
        pl.BlockSpec(a.shape, lambda i: tuple(0 for _ in a.shape))
        for a in full_ins
    ]
    out_shape = [jax.ShapeDtypeStruct((R, d), jnp.float32) for d in out_dims]
    out_specs = [pl.BlockSpec((br, d), lambda i: (i, 0)) for d in out_dims]
    res = pl.pallas_call(
        kern,
        grid=(R // br,),
        in_specs=in_specs,
        out_specs=out_specs,
        out_shape=out_shape,
    )(*row_ins, *full_ins)
    return res


def _ln(x, g, b):
    mu = jnp.mean(x, axis=-1, keepdims=True)
    var = jnp.mean((x - mu) ** 2, axis=-1, keepdims=True)
    return g * (x - mu) * lax.rsqrt(var + 1e-5) + b


def _mm(x, w):
    return jnp.dot(x, w, preferred_element_type=jnp.float32)


def _pb(p):
    return (p["W"], p["b"][None, :], p["g"][None, :], p["be"][None, :])


def _proj(x, W, b, g, be):
    return jax.nn.relu(_ln(_mm(x, W) + b, g, be))


def kernel(node_feats, nodes_of_lanes, l2l_encoder_edges, l2l_encoder_rpes, params):
    NP, DIN = node_feats.shape
    E = l2l_encoder_rpes.shape[0]
    NL = 10000

    src = l2l_encoder_edges[0]
    dst = l2l_encoder_edges[1]
    perm = jnp.argsort(dst).astype(jnp.int32)
    dst_s = dst[perm]
    src_s = src[perm]
    offs_e = jnp.searchsorted(dst_s, jnp.arange(NL + 1, dtype=jnp.int32)).astype(jnp.int32)
    offs_p = jnp.searchsorted(nodes_of_lanes, jnp.arange(NL + 1, dtype=jnp.int32)).astype(jnp.int32)

    NP_pad = ((NP + 2559) // 2560) * 2560
    ids_pad = jnp.pad(nodes_of_lanes.astype(jnp.int32), (0, NP_pad - NP))

    nf16 = jnp.pad(node_feats, ((0, 0), (0, 16 - DIN)))
    rp16 = jnp.pad(l2l_encoder_rpes, ((0, 0), (0, 16 - l2l_encoder_rpes.shape[1])))
    rp16_s = rp16[perm]

    P = params
    hmask = (jnp.arange(_D)[:, None] // _DH == jnp.arange(_H)[None, :]).astype(jnp.float32)
    hmask_t = hmask.T

    Wp = jnp.pad(P["proj"]["W"], ((0, 16 - DIN), (0, 0)))
    Wr = jnp.pad(P["rpe"]["W"], ((0, 16 - l2l_encoder_rpes.shape[1]), (0, 0)))

    pa = P["point_aggr"]

    def p1_body(row_vals, fv):
        (nf,) = row_vals
        Wp_, bp, gp, bep, W1a, b1a, g1a, be1a, W1b, b1b, g1b, be1b = fv
        x0 = _proj(nf, Wp_, bp, gp, bep)
        p1 = _proj(x0, W1a, b1a, g1a, be1a)
        p2 = _proj(p1, W1b, b1b, g1b, be1b)
        return x0, p2

    x0, p2 = _rows(
        p1_body, [nf16],
        [Wp, P["proj"]["b"][None], P["proj"]["g"][None], P["proj"]["be"][None],
         *_pb(pa["fc1a"]), *_pb(pa["fc1b"])],
        [_D, _D])

    mx = _seg_reduce(offs_p, p2, "max", NL)
    mxg = _sc_gather(mx, ids_pad)[:NP]

    la = P["lane_aggr"]

    def p2_body(row_vals, fv):
        p2v, mxv, x0v = row_vals
        (W2a, b2a, g2a, be2a, W2b, b2b, g2b, be2b, ng, nb,
         W1a, b1a, g1a, be1a, W1b, b1b, g1b, be1b) = fv
        y = jax.nn.relu(_ln(_mm(p2v, W2a[:_D]) + _mm(mxv, W2a[_D:]) + b2a, g2a, be2a))
        y = _proj(y, W2b, b2b, g2b, be2b)
        out1 = _ln(x0v + y, ng, nb)
        q1 = _proj(out1, W1a, b1a, g1a, be1a)
        q2 = _proj(q1, W1b, b1b, g1b, be1b)
        return out1, q2

    out1, q2 = _rows(
        p2_body, [p2, mxg, x0],
        [*_pb(pa["fc2a"]), *_pb(pa["fc2b"]), pa["ng"][None], pa["nb"][None],
         *_pb(la["fc1a"]), *_pb(la["fc1b"])],
        [_D, _D])

    mx2 = _seg_reduce(offs_p, q2, "max", NL)
    mxg2 = _sc_gather(mx2, ids_pad)[:NP]

    def p3_body(row_vals, fv):
        q2v, mxv, out1v = row_vals
        (W2a, b2a, g2a, be2a, W2b, b2b, g2b, be2b, ng, nb) = fv
        y = jax.nn.relu(_ln(_mm(q2v, W2a[:_D]) + _mm(mxv, W2a[_D:]) + b2a, g2a, be2a))
        y = _proj(y, W2b, b2b, g2b, be2b)
        return _ln(out1v + y, ng, nb)

    (out2,) = _rows(
        p3_body, [q2, mxg2, out1],
        [*_pb(la["fc2a"]), *_pb(la["fc2b"]), la["ng"][None], la["nb"][None]],
        [_D])

    lane = _seg_reduce(offs_p, out2, "max", NL)

    def e0_body(row_vals, fv):
        (rp,) = row_vals
        Wr_, br_, gr, ber = fv
        return _proj(rp, Wr_, br_, gr, ber)

    (ea,) = _rows(
        e0_body, [rp16_s],
        [Wr, P["rpe"]["b"][None], P["rpe"]["g"][None], P["rpe"]["be"][None]],
        [_D])

    for lp in P["layers"]:
        Wm = lp["mem"]["W"]
        Wm_i, Wm_j, Wm_e = Wm[:_D], Wm[_D:2 * _D], Wm[2 * _D:]

        def lpre_body(row_vals, fv):
            (lv,) = row_vals
            wmi, wmj, qw = fv
            aq = jnp.concatenate([_mm(lv, wmi), _mm(lv, qw)], axis=1)
            bm = _mm(lv, wmj)
            return aq, bm

        AQ, Bm = _rows(lpre_body, [lane], [Wm_i, Wm_j, lp["qW"]], [2 * _D, _D])

        AQg = _sc_gather(AQ, dst_s)
        Bg = _sc_gather(Bm, src_s)

        def e1_body(row_vals, fv):
            aq, bg, eav = row_vals
            (Wm_e_, bm_, gm, bem, euW, eub, geu, beu, eng, enb,
             kW, vW, hm, hmt) = fv
            pre = aq[:, :_D] + bg + _mm(eav, Wm_e_) + bm_
            mem = jax.nn.relu(_ln(pre, gm, bem))
            du = jax.nn.relu(_ln(_mm(mem, euW) + eub, geu, beu))
            ean = _ln(eav + du, eng, enb)
            k = _mm(mem, kW)
            v = _mm(mem, vW)
            q = aq[:, _D:]
            logits = _mm(q * k, hm) * 0.25
            ex = jnp.exp(logits)
            w = v * _mm(ex, hmt)
            return ean, w, ex

        mb = lp["mem"]
        eu = lp["eu"]
        ea, w, ex = _rows(
            e1_body, [AQg, Bg, ea],
            [Wm_e, mb["b"][None], mb["g"][None], mb["be"][None],
             eu["W"], eu["b"][None], eu["g"][None], eu["be"][None],
             lp["eng"][None], lp["enb"][None],
             lp["kW"], lp["vW"], hmask, hmask_t],
            [_D, _D, _H])

        wsum = _seg_reduce(offs_e, w, "sum", NL)
        s = _seg_reduce(offs_e, ex, "sum", NL)

        def lupd_body(row_vals, fv):
            lv, wsv, sv = row_vals
            (oW, n1g, n1b, f1W, f1b, f2W, f2b, n2g, n2b, hmt) = fv
            inv = 1.0 / (sv + 1e-16)
            agg = _mm(wsv * _mm(inv, hmt), oW)
            x = _ln(lv + agg, n1g, n1b)
            ff = _mm(jax.nn.relu(_mm(x, f1W) + f1b), f2W) + f2b
            return _ln(x + ff, n2g, n2b)

        (lane,) = _rows(
            lupd_body, [lane, wsum, s],
            [lp["oW"], lp["n1g"][None], lp["n1b"][None],
             lp["f1W"], lp["f1b"][None], lp["f2W"], lp["f2b"][None],
             lp["n2g"][None], lp["n2b"][None], hmask_t],
            [_D])

    return lane


# speedup vs baseline: 1.0170x; 1.0170x over previous
import functools

import jax
import jax.numpy as jnp
from jax import lax
from jax.experimental import pallas as pl
from jax.experimental.pallas import tpu as pltpu
from jax.experimental.pallas import tpu_sc as plsc

_D = 128
_H = 8
_DH = _D // _H


def _sc_gather(table, idx, chunk=80):
    V, D = table.shape
    B = idx.shape[0]
    info = plsc.get_sparse_core_info()
    nc, ns = info.num_cores, info.num_subcores
    nw = nc * ns
    assert B % (nw * chunk) == 0, (B, nw, chunk)
    b_per_w = B // nw
    nchunks = b_per_w // chunk
    mesh = plsc.VectorSubcoreMesh(core_axis_name="c", subcore_axis_name="s")

    def body(table_hbm, idx_hbm, out_hbm, idx_v, rows_v, sem):
        wid = lax.axis_index("s") * nc + lax.axis_index("c")
        base = wid * b_per_w

        @pl.loop(0, nchunks)
        def _(t):
            off = base + t * chunk
            pltpu.sync_copy(idx_hbm.at[pl.ds(off, chunk)], idx_v)
            pltpu.async_copy(table_hbm.at[idx_v], rows_v, sem).wait()
            pltpu.sync_copy(rows_v, out_hbm.at[pl.ds(off, chunk)])

    return pl.kernel(
        body,
        out_type=jax.ShapeDtypeStruct((B, D), table.dtype),
        mesh=mesh,
        scratch_types=[
            pltpu.VMEM((chunk,), jnp.int32),
            pltpu.VMEM((chunk, D), table.dtype),
            pltpu.SemaphoreType.DMA,
        ],
    )(table, idx)


def _seg_reduce(offsets, vals_list, mode, n_seg, bl, c=512):
    nv = len(vals_list)
    R = vals_list[0].shape[0]
    dims = [v.shape[1] for v in vals_list]
    assert n_seg % bl == 0

    def kern(off_ref, *refs):
        vals_hbms = refs[:nv]
        out_refs = refs[nv:2 * nv]
        scratch = refs[2 * nv:3 * nv]
        sems = refs[3 * nv]
        b = pl.program_id(0)
        lane0 = b * bl
        start = off_ref[lane0]
        end = off_ref[lane0 + bl]
        nch = (end - start + c - 1) // c

        if mode == "max":
            inits = [jnp.full((bl, d), -jnp.inf, jnp.float32) for d in dims]
        else:
            inits = [jnp.zeros((bl, d), jnp.float32) for d in dims]

        def chunk_body(t, accs):
            off_i = start + t * c
            off_r = jnp.minimum(off_i, R - c)
            cps = [
                pltpu.make_async_copy(
                    vals_hbms[i].at[pl.ds(off_r, c), :], scratch[i], sems)
                for i in range(nv)
            ]
            for cp in cps:
                cp.start()
            for cp in cps:
                cp.wait()
            gidx = off_r + lax.broadcasted_iota(jnp.int32, (c, 1), 0)
            valid = gidx >= off_i
            vs = [scratch[i][...] for i in range(nv)]
            if mode == "max":
                new = []
                for i in range(nv):
                    rows = []
                    for j in range(bl):
                        sj = off_ref[lane0 + j]
                        ej = off_ref[lane0 + j + 1]
                        m = (gidx >= sj) & (gidx < ej) & valid
                        contrib = jnp.where(m, vs[i], -jnp.inf).max(
                            axis=0, keepdims=True)
                        rows.append(jnp.maximum(accs[i][j:j + 1], contrib))
                    new.append(jnp.concatenate(rows, axis=0))
                return tuple(new)
            else:
                cols = []
                for j in range(bl):
                    sj = off_ref[lane0 + j]
                    ej = off_ref[lane0 + j + 1]
                    m = (gidx >= sj) & (gidx < ej) & valid
                    cols.append(m.astype(jnp.float32))
                mask = jnp.concatenate(cols, axis=1)
                new = []
                for i in range(nv):
                    part = lax.dot_general(
                        mask, vs[i], (((0,), (0,)), ((), ())),
                        preferred_element_type=jnp.float32)
                    new.append(accs[i] + part)
                return tuple(new)

        accs = lax.fori_loop(0, nch, chunk_body, tuple(inits))
        for i in range(nv):
            a = accs[i]
            if mode == "max":
                a = jnp.where(a == -jnp.inf, 0.0, a)
            out_refs[i][...] = a

    grid_spec = pltpu.PrefetchScalarGridSpec(
        num_scalar_prefetch=1,
        grid=(n_seg // bl,),
        in_specs=[pl.BlockSpec(memory_space=pl.MemorySpace.ANY)] * nv,
        out_specs=[pl.BlockSpec((bl, d), lambda b, off: (b, 0)) for d in dims],
        scratch_shapes=[pltpu.VMEM((c, d), jnp.float32) for d in dims]
        + [pltpu.SemaphoreType.DMA],
    )
    res = pl.pallas_call(
        kern,
        grid_spec=grid_spec,
        out_shape=[jax.ShapeDtypeStruct((n_seg, d), jnp.float32) for d in dims],
    )(offsets, *vals_list)
    return res


def _rows(body, row_ins, full_ins, out_dims, br=1000):
    R = row_ins[0].shape[0]
    assert R % br == 0
    n_row = len(row_ins)

    def kern(*refs):
        ins = refs[: n_row + len(full_ins)]
        outs = refs[n_row + len(full_ins):]
        row_vals = [r[...] for r in ins[:n_row]]
        full_vals = [r[...] for r in ins[n_row:]]
        res = body(row_vals, full_vals)
        if not isinstance(res, tuple):
            res = (res,)
        for o_ref, r in zip(outs, res):
            o_ref[...] = r

    in_specs = [
        pl.BlockSpec((br, a.shape[1]), lambda i: (i, 0)) for a in row_ins
    ] + [
        pl.BlockSpec(a.shape, lambda i: tuple(0 for _ in a.shape))
        for a in full_ins
    ]
    out_shape = [jax.ShapeDtypeStruct((R, d), jnp.float32) for d in out_dims]
    out_specs = [pl.BlockSpec((br, d), lambda i: (i, 0)) for d in out_dims]
    res = pl.pallas_call(
        kern,
        grid=(R // br,),
        in_specs=in_specs,
        out_specs=out_specs,
        out_shape=out_shape,
    )(*row_ins, *full_ins)
    return res


def _ln(x, g, b):
    mu = jnp.mean(x, axis=-1, keepdims=True)
    var = jnp.mean((x - mu) ** 2, axis=-1, keepdims=True)
    return g * (x - mu) * lax.rsqrt(var + 1e-5) + b


def _mm(x, w):
    return jnp.dot(x, w, preferred_element_type=jnp.float32)


def _pb(p):
    return (p["W"], p["b"][None, :], p["g"][None, :], p["be"][None, :])


def _proj(x, W, b, g, be):
    return jax.nn.relu(_ln(_mm(x, W) + b, g, be))


def kernel(node_feats, nodes_of_lanes, l2l_encoder_edges, l2l_encoder_rpes, params):
    NP, DIN = node_feats.shape
    E = l2l_encoder_rpes.shape[0]
    NL = 10000

    src = l2l_encoder_edges[0]
    dst = l2l_encoder_edges[1]
    perm = jnp.argsort(dst).astype(jnp.int32)
    dst_s = dst[perm]
    src_s = src[perm]
    offs_e = jnp.searchsorted(dst_s, jnp.arange(NL + 1, dtype=jnp.int32)).astype(jnp.int32)
    offs_p = jnp.searchsorted(nodes_of_lanes, jnp.arange(NL + 1, dtype=jnp.int32)).astype(jnp.int32)

    NP_pad = ((NP + 2559) // 2560) * 2560
    ids_pad = jnp.pad(nodes_of_lanes.astype(jnp.int32), (0, NP_pad - NP))

    nf16 = jnp.pad(node_feats, ((0, 0), (0, 16 - DIN)))
    rp16 = jnp.pad(l2l_encoder_rpes, ((0, 0), (0, 16 - l2l_encoder_rpes.shape[1])))
    rp16_s = rp16[perm]

    P = params
    hmask = (jnp.arange(_D)[:, None] // _DH == jnp.arange(_H)[None, :]).astype(jnp.float32)
    hmask_t = hmask.T

    Wp = jnp.pad(P["proj"]["W"], ((0, 16 - DIN), (0, 0)))
    Wr = jnp.pad(P["rpe"]["W"], ((0, 16 - l2l_encoder_rpes.shape[1]), (0, 0)))

    pa = P["point_aggr"]

    def p1_body(row_vals, fv):
        (nf,) = row_vals
        Wp_, bp, gp, bep, W1a, b1a, g1a, be1a, W1b, b1b, g1b, be1b = fv
        x0 = _proj(nf, Wp_, bp, gp, bep)
        p1 = _proj(x0, W1a, b1a, g1a, be1a)
        p2 = _proj(p1, W1b, b1b, g1b, be1b)
        return x0, p2

    x0, p2 = _rows(
        p1_body, [nf16],
        [Wp, P["proj"]["b"][None], P["proj"]["g"][None], P["proj"]["be"][None],
         *_pb(pa["fc1a"]), *_pb(pa["fc1b"])],
        [_D, _D])

    (mx,) = _seg_reduce(offs_p, [p2], "max", NL, bl=40)
    mxg = _sc_gather(mx, ids_pad)[:NP]

    la = P["lane_aggr"]

    def p2_body(row_vals, fv):
        p2v, mxv, x0v = row_vals
        (W2a, b2a, g2a, be2a, W2b, b2b, g2b, be2b, ng, nb,
         W1a, b1a, g1a, be1a, W1b, b1b, g1b, be1b) = fv
        y = jax.nn.relu(_ln(_mm(p2v, W2a[:_D]) + _mm(mxv, W2a[_D:]) + b2a, g2a, be2a))
        y = _proj(y, W2b, b2b, g2b, be2b)
        out1 = _ln(x0v + y, ng, nb)
        q1 = _proj(out1, W1a, b1a, g1a, be1a)
        q2 = _proj(q1, W1b, b1b, g1b, be1b)
        return out1, q2

    out1, q2 = _rows(
        p2_body, [p2, mxg, x0],
        [*_pb(pa["fc2a"]), *_pb(pa["fc2b"]), pa["ng"][None], pa["nb"][None],
         *_pb(la["fc1a"]), *_pb(la["fc1b"])],
        [_D, _D])

    (mx2,) = _seg_reduce(offs_p, [q2], "max", NL, bl=40)
    mxg2 = _sc_gather(mx2, ids_pad)[:NP]

    def p3_body(row_vals, fv):
        q2v, mxv, out1v = row_vals
        (W2a, b2a, g2a, be2a, W2b, b2b, g2b, be2b, ng, nb) = fv
        y = jax.nn.relu(_ln(_mm(q2v, W2a[:_D]) + _mm(mxv, W2a[_D:]) + b2a, g2a, be2a))
        y = _proj(y, W2b, b2b, g2b, be2b)
        return _ln(out1v + y, ng, nb)

    (out2,) = _rows(
        p3_body, [q2, mxg2, out1],
        [*_pb(la["fc2a"]), *_pb(la["fc2b"]), la["ng"][None], la["nb"][None]],
        [_D])

    (lane,) = _seg_reduce(offs_p, [out2], "max", NL, bl=40)

    def e0_body(row_vals, fv):
        (rp,) = row_vals
        Wr_, br_, gr, ber = fv
        return _proj(rp, Wr_, br_, gr, ber)

    (ea,) = _rows(
        e0_body, [rp16_s],
        [Wr, P["rpe"]["b"][None], P["rpe"]["g"][None], P["rpe"]["be"][None]],
        [_D])

    for lp in P["layers"]:
        Wm = lp["mem"]["W"]
        Wm_i, Wm_j, Wm_e = Wm[:_D], Wm[_D:2 * _D], Wm[2 * _D:]

        def lpre_body(row_vals, fv):
            (lv,) = row_vals
            wmi, wmj, qw = fv
            aq = jnp.concatenate([_mm(lv, wmi), _mm(lv, qw)], axis=1)
            bm = _mm(lv, wmj)
            return aq, bm

        AQ, Bm = _rows(lpre_body, [lane], [Wm_i, Wm_j, lp["qW"]], [2 * _D, _D])

        AQg = _sc_gather(AQ, dst_s)
        Bg = _sc_gather(Bm, src_s)

        def e1_body(row_vals, fv):
            aq, bg, eav = row_vals
            (Wm_e_, bm_, gm, bem, euW, eub, geu, beu, eng, enb,
             kW, vW, hm, hmt) = fv
            pre = aq[:, :_D] + bg + _mm(eav, Wm_e_) + bm_
            mem = jax.nn.relu(_ln(pre, gm, bem))
            du = jax.nn.relu(_ln(_mm(mem, euW) + eub, geu, beu))
            ean = _ln(eav + du, eng, enb)
            k = _mm(mem, kW)
            v = _mm(mem, vW)
            q = aq[:, _D:]
            logits = _mm(q * k, hm) * 0.25
            ex = jnp.exp(logits)
            w = v * _mm(ex, hmt)
            return ean, w, ex

        mb = lp["mem"]
        eu = lp["eu"]
        ea, w, ex = _rows(
            e1_body, [AQg, Bg, ea],
            [Wm_e, mb["b"][None], mb["g"][None], mb["be"][None],
             eu["W"], eu["b"][None], eu["g"][None], eu["be"][None],
             lp["eng"][None], lp["enb"][None],
             lp["kW"], lp["vW"], hmask, hmask_t],
            [_D, _D, _H])

        wsum, s = _seg_reduce(offs_e, [w, ex], "sum", NL, bl=80)

        def lupd_body(row_vals, fv):
            lv, wsv, sv = row_vals
            (oW, n1g, n1b, f1W, f1b, f2W, f2b, n2g, n2b, hmt) = fv
            inv = 1.0 / (sv + 1e-16)
            agg = _mm(wsv * _mm(inv, hmt), oW)
            x = _ln(lv + agg, n1g, n1b)
            ff = _mm(jax.nn.relu(_mm(x, f1W) + f1b), f2W) + f2b
            return _ln(x + ff, n2g, n2b)

        (lane,) = _rows(
            lupd_body, [lane, wsum, s],
            [lp["oW"], lp["n1g"][None], lp["n1b"][None],
             lp["f1W"], lp["f1b"][None], lp["f2W"], lp["f2b"][None],
             lp["n2g"][None], lp["n2b"][None], hmask_t],
            [_D])

    return lane


# speedup vs baseline: 1.0403x; 1.0229x over previous
import functools

import jax
import jax.numpy as jnp
from jax import lax
from jax.experimental import pallas as pl
from jax.experimental.pallas import tpu as pltpu
from jax.experimental.pallas import tpu_sc as plsc

_D = 128
_H = 8
_DH = _D // _H


def _sc_gather(table, idx, chunk=200):
    V, D = table.shape
    B = idx.shape[0]
    info = plsc.get_sparse_core_info()
    nc, ns = info.num_cores, info.num_subcores
    nw = nc * ns
    assert B % (nw * chunk) == 0, (B, nw, chunk)
    b_per_w = B // nw
    nchunks = b_per_w // chunk
    mesh = plsc.VectorSubcoreMesh(core_axis_name="c", subcore_axis_name="s")

    def body(table_hbm, idx_hbm, out_hbm, idx_v, rows_v, sem):
        wid = lax.axis_index("s") * nc + lax.axis_index("c")
        base = wid * b_per_w

        @pl.loop(0, nchunks)
        def _(t):
            off = base + t * chunk
            pltpu.sync_copy(idx_hbm.at[pl.ds(off, chunk)], idx_v)
            pltpu.async_copy(table_hbm.at[idx_v], rows_v, sem).wait()
            pltpu.sync_copy(rows_v, out_hbm.at[pl.ds(off, chunk)])

    return pl.kernel(
        body,
        out_type=jax.ShapeDtypeStruct((B, D), table.dtype),
        mesh=mesh,
        scratch_types=[
            pltpu.VMEM((chunk,), jnp.int32),
            pltpu.VMEM((chunk, D), table.dtype),
            pltpu.SemaphoreType.DMA,
        ],
    )(table, idx)


def _seg_reduce(offsets, vals_list, mode, n_seg, bl, c=512):
    nv = len(vals_list)
    R = vals_list[0].shape[0]
    dims = [v.shape[1] for v in vals_list]
    assert n_seg % bl == 0

    def kern(off_ref, *refs):
        vals_hbms = refs[:nv]
        out_refs = refs[nv:2 * nv]
        scratch = refs[2 * nv:3 * nv]
        sems = refs[3 * nv]
        b = pl.program_id(0)
        lane0 = b * bl
        start = off_ref[lane0]
        end = off_ref[lane0 + bl]
        nch = (end - start + c - 1) // c

        if mode == "max":
            inits = [jnp.full((bl, d), -jnp.inf, jnp.float32) for d in dims]
        else:
            inits = [jnp.zeros((bl, d), jnp.float32) for d in dims]

        def chunk_body(t, accs):
            off_i = start + t * c
            off_r = jnp.minimum(off_i, R - c)
            cps = [
                pltpu.make_async_copy(
                    vals_hbms[i].at[pl.ds(off_r, c), :], scratch[i], sems)
                for i in range(nv)
            ]
            for cp in cps:
                cp.start()
            for cp in cps:
                cp.wait()
            gidx = off_r + lax.broadcasted_iota(jnp.int32, (c, 1), 0)
            valid = gidx >= off_i
            vs = [scratch[i][...] for i in range(nv)]
            if mode == "max":
                new = []
                for i in range(nv):
                    rows = []
                    for j in range(bl):
                        sj = off_ref[lane0 + j]
                        ej = off_ref[lane0 + j + 1]
                        m = (gidx >= sj) & (gidx < ej) & valid
                        contrib = jnp.where(m, vs[i], -jnp.inf).max(
                            axis=0, keepdims=True)
                        rows.append(jnp.maximum(accs[i][j:j + 1], contrib))
                    new.append(jnp.concatenate(rows, axis=0))
                return tuple(new)
            else:
                cols = []
                for j in range(bl):
                    sj = off_ref[lane0 + j]
                    ej = off_ref[lane0 + j + 1]
                    m = (gidx >= sj) & (gidx < ej) & valid
                    cols.append(m.astype(jnp.float32))
                mask = jnp.concatenate(cols, axis=1)
                new = []
                for i in range(nv):
                    part = lax.dot_general(
                        mask, vs[i], (((0,), (0,)), ((), ())),
                        preferred_element_type=jnp.float32)
                    new.append(accs[i] + part)
                return tuple(new)

        accs = lax.fori_loop(0, nch, chunk_body, tuple(inits))
        for i in range(nv):
            a = accs[i]
            if mode == "max":
                a = jnp.where(a == -jnp.inf, 0.0, a)
            out_refs[i][...] = a

    grid_spec = pltpu.PrefetchScalarGridSpec(
        num_scalar_prefetch=1,
        grid=(n_seg // bl,),
        in_specs=[pl.BlockSpec(memory_space=pl.MemorySpace.ANY)] * nv,
        out_specs=[pl.BlockSpec((bl, d), lambda b, off: (b, 0)) for d in dims],
        scratch_shapes=[pltpu.VMEM((c, d), jnp.float32) for d in dims]
        + [pltpu.SemaphoreType.DMA],
    )
    res = pl.pallas_call(
        kern,
        grid_spec=grid_spec,
        out_shape=[jax.ShapeDtypeStruct((n_seg, d), jnp.float32) for d in dims],
    )(offsets, *vals_list)
    return res


def _rows(body, row_ins, full_ins, out_dims, br=1000):
    R = row_ins[0].shape[0]
    assert R % br == 0
    n_row = len(row_ins)

    def kern(*refs):
        ins = refs[: n_row + len(full_ins)]
        outs = refs[n_row + len(full_ins):]
        row_vals = [r[...] for r in ins[:n_row]]
        full_vals = [r[...] for r in ins[n_row:]]
        res = body(row_vals, full_vals)
        if not isinstance(res, tuple):
            res = (res,)
        for o_ref, r in zip(outs, res):
            o_ref[...] = r

    in_specs = [
        pl.BlockSpec((br, a.shape[1]), lambda i: (i, 0)) for a in row_ins
    ] + [
        pl.BlockSpec(a.shape, lambda i: tuple(0 for _ in a.shape))
        for a in full_ins
    ]
    out_shape = [jax.ShapeDtypeStruct((R, d), jnp.float32) for d in out_dims]
    out_specs = [pl.BlockSpec((br, d), lambda i: (i, 0)) for d in out_dims]
    res = pl.pallas_call(
        kern,
        grid=(R // br,),
        in_specs=in_specs,
        out_specs=out_specs,
        out_shape=out_shape,
    )(*row_ins, *full_ins)
    return res


def _ln(x, g, b):
    mu = jnp.mean(x, axis=-1, keepdims=True)
    var = jnp.mean((x - mu) ** 2, axis=-1, keepdims=True)
    return g * (x - mu) * lax.rsqrt(var + 1e-5) + b


def _mm(x, w):
    return jnp.dot(x, w, preferred_element_type=jnp.float32)


def _pb(p):
    return (p["W"], p["b"][None, :], p["g"][None, :], p["be"][None, :])


def _proj(x, W, b, g, be):
    return jax.nn.relu(_ln(_mm(x, W) + b, g, be))


def kernel(node_feats, nodes_of_lanes, l2l_encoder_edges, l2l_encoder_rpes, params):
    NP, DIN = node_feats.shape
    E = l2l_encoder_rpes.shape[0]
    NL = 10000

    src = l2l_encoder_edges[0]
    dst = l2l_encoder_edges[1]
    perm = jnp.argsort(dst).astype(jnp.int32)
    dst_s = dst[perm]
    src_s = src[perm]
    offs_e = jnp.searchsorted(dst_s, jnp.arange(NL + 1, dtype=jnp.int32)).astype(jnp.int32)
    offs_p = jnp.searchsorted(nodes_of_lanes, jnp.arange(NL + 1, dtype=jnp.int32)).astype(jnp.int32)

    NP_pad = ((NP + 2559) // 2560) * 2560
    ids_pad = jnp.pad(nodes_of_lanes.astype(jnp.int32), (0, NP_pad - NP))

    nf16 = jnp.pad(node_feats, ((0, 0), (0, 16 - DIN)))
    rp16 = jnp.pad(l2l_encoder_rpes, ((0, 0), (0, 16 - l2l_encoder_rpes.shape[1])))
    rp16_s = rp16[perm]

    P = params
    hmask = (jnp.arange(_D)[:, None] // _DH == jnp.arange(_H)[None, :]).astype(jnp.float32)
    hmask_t = hmask.T

    Wp = jnp.pad(P["proj"]["W"], ((0, 16 - DIN), (0, 0)))
    Wr = jnp.pad(P["rpe"]["W"], ((0, 16 - l2l_encoder_rpes.shape[1]), (0, 0)))

    pa = P["point_aggr"]

    def p1_body(row_vals, fv):
        (nf,) = row_vals
        Wp_, bp, gp, bep, W1a, b1a, g1a, be1a, W1b, b1b, g1b, be1b = fv
        x0 = _proj(nf, Wp_, bp, gp, bep)
        p1 = _proj(x0, W1a, b1a, g1a, be1a)
        p2 = _proj(p1, W1b, b1b, g1b, be1b)
        return x0, p2

    x0, p2 = _rows(
        p1_body, [nf16],
        [Wp, P["proj"]["b"][None], P["proj"]["g"][None], P["proj"]["be"][None],
         *_pb(pa["fc1a"]), *_pb(pa["fc1b"])],
        [_D, _D])

    (mx,) = _seg_reduce(offs_p, [p2], "max", NL, bl=40)
    mxg = _sc_gather(mx, ids_pad)[:NP]

    la = P["lane_aggr"]

    def p2_body(row_vals, fv):
        p2v, mxv, x0v = row_vals
        (W2a, b2a, g2a, be2a, W2b, b2b, g2b, be2b, ng, nb,
         W1a, b1a, g1a, be1a, W1b, b1b, g1b, be1b) = fv
        y = jax.nn.relu(_ln(_mm(p2v, W2a[:_D]) + _mm(mxv, W2a[_D:]) + b2a, g2a, be2a))
        y = _proj(y, W2b, b2b, g2b, be2b)
        out1 = _ln(x0v + y, ng, nb)
        q1 = _proj(out1, W1a, b1a, g1a, be1a)
        q2 = _proj(q1, W1b, b1b, g1b, be1b)
        return out1, q2

    out1, q2 = _rows(
        p2_body, [p2, mxg, x0],
        [*_pb(pa["fc2a"]), *_pb(pa["fc2b"]), pa["ng"][None], pa["nb"][None],
         *_pb(la["fc1a"]), *_pb(la["fc1b"])],
        [_D, _D])

    (mx2,) = _seg_reduce(offs_p, [q2], "max", NL, bl=40)
    mxg2 = _sc_gather(mx2, ids_pad)[:NP]

    def p3_body(row_vals, fv):
        q2v, mxv, out1v = row_vals
        (W2a, b2a, g2a, be2a, W2b, b2b, g2b, be2b, ng, nb) = fv
        y = jax.nn.relu(_ln(_mm(q2v, W2a[:_D]) + _mm(mxv, W2a[_D:]) + b2a, g2a, be2a))
        y = _proj(y, W2b, b2b, g2b, be2b)
        return _ln(out1v + y, ng, nb)

    (out2,) = _rows(
        p3_body, [q2, mxg2, out1],
        [*_pb(la["fc2a"]), *_pb(la["fc2b"]), la["ng"][None], la["nb"][None]],
        [_D])

    (lane,) = _seg_reduce(offs_p, [out2], "max", NL, bl=40)

    def e0_body(row_vals, fv):
        (rp,) = row_vals
        Wr_, br_, gr, ber = fv
        return _proj(rp, Wr_, br_, gr, ber)

    (ea,) = _rows(
        e0_body, [rp16_s],
        [Wr, P["rpe"]["b"][None], P["rpe"]["g"][None], P["rpe"]["be"][None]],
        [_D])

    for lp in P["layers"]:
        Wm = lp["mem"]["W"]
        Wm_i, Wm_j, Wm_e = Wm[:_D], Wm[_D:2 * _D], Wm[2 * _D:]

        def lpre_body(row_vals, fv):
            (lv,) = row_vals
            wmi, wmj, qw = fv
            aq = jnp.concatenate([_mm(lv, wmi), _mm(lv, qw)], axis=1)
            bm = _mm(lv, wmj)
            return aq, bm

        AQ, Bm = _rows(lpre_body, [lane], [Wm_i, Wm_j, lp["qW"]], [2 * _D, _D])

        AQg = _sc_gather(AQ, dst_s)
        Bg = _sc_gather(Bm, src_s)

        def e1_body(row_vals, fv):
            aq, bg, eav = row_vals
            (Wm_e_, bm_, gm, bem, euW, eub, geu, beu, eng, enb,
             kW, vW, hm, hmt) = fv
            pre = aq[:, :_D] + bg + _mm(eav, Wm_e_) + bm_
            mem = jax.nn.relu(_ln(pre, gm, bem))
            du = jax.nn.relu(_ln(_mm(mem, euW) + eub, geu, beu))
            ean = _ln(eav + du, eng, enb)
            k = _mm(mem, kW)
            v = _mm(mem, vW)
            q = aq[:, _D:]
            logits = _mm(q * k, hm) * 0.25
            ex = jnp.exp(logits)
            w = v * _mm(ex, hmt)
            return ean, w, ex

        mb = lp["mem"]
        eu = lp["eu"]
        ea, w, ex = _rows(
            e1_body, [AQg, Bg, ea],
            [Wm_e, mb["b"][None], mb["g"][None], mb["be"][None],
             eu["W"], eu["b"][None], eu["g"][None], eu["be"][None],
             lp["eng"][None], lp["enb"][None],
             lp["kW"], lp["vW"], hmask, hmask_t],
            [_D, _D, _H])

        wsum, s = _seg_reduce(offs_e, [w, ex], "sum", NL, bl=80)

        def lupd_body(row_vals, fv):
            lv, wsv, sv = row_vals
            (oW, n1g, n1b, f1W, f1b, f2W, f2b, n2g, n2b, hmt) = fv
            inv = 1.0 / (sv + 1e-16)
            agg = _mm(wsv * _mm(inv, hmt), oW)
            x = _ln(lv + agg, n1g, n1b)
            ff = _mm(jax.nn.relu(_mm(x, f1W) + f1b), f2W) + f2b
            return _ln(x + ff, n2g, n2b)

        (lane,) = _rows(
            lupd_body, [lane, wsum, s],
            [lp["oW"], lp["n1g"][None], lp["n1b"][None],
             lp["f1W"], lp["f1b"][None], lp["f2W"], lp["f2b"][None],
             lp["n2g"][None], lp["n2b"][None], hmask_t],
            [_D])

    return lane
